# baseline (speedup 1.0000x reference)
import math

import jax
import jax.numpy as jnp
from jax import lax
from jax.experimental import pallas as pl

_BB = 64
_SG = 4
_N_STEPS = 8


def _gelu_y(y):
    q = y * math.sqrt(0.5)
    return q + q * lax.erf(y)


def _gnn_block_kernel(es_ref, er_ref, et_ref, ne_ref, qs_ref, qt_ref,
                      ent_ref, rel_ref,
                      mw1_ref, mb1_ref, mw2_ref, mb2_ref,
                      uw1_ref, ub1_ref, uw2_ref, ub2_ref,
                      cw1_ref, cb1_ref, cw2_ref, cb2_ref,
                      out_ref):
    n_ent, d = ent_ref.shape
    max_e = es_ref.shape[2]
    n_rel = rel_ref.shape[0]
    bb = _BB
    sg = _SG
    nsub = bb // sg
    srows = sg * max_e
    scols = sg * n_ent
    rows = bb * max_e
    cols = bb * n_ent

    rs2 = math.sqrt(0.5)
    w1 = mw1_ref[...] * rs2
    w1a = w1[:d, :]
    w1b = w1[d:2 * d, :]
    w1c = w1[2 * d:, :]
    mb1 = mb1_ref[...] * rs2
    mw2 = mw2_ref[...]
    mb2 = mb2_ref[...]
    uw1 = uw1_ref[...] * rs2
    u1a = uw1[:d, :]
    u1b = uw1[d:, :]
    ub1 = ub1_ref[...] * rs2
    uw2 = uw2_ref[...]
    ub2 = ub2_ref[...]

    es = es_ref[0]
    et = et_ref[0]
    er = er_ref[0]
    ne = ne_ref[0]
    eidx = lax.broadcasted_iota(jnp.int32, (bb, max_e), 1)
    mask = (eidx < ne).astype(jnp.float32)
    base = (lax.broadcasted_iota(jnp.int32, (bb, max_e), 0) % sg) * n_ent
    col3 = lax.broadcasted_iota(jnp.int32, (bb, max_e, scols), 2)
    g_src = ((col3 == (es + base)[:, :, None]).astype(jnp.float32)
             * mask[:, :, None]).reshape(nsub, srows, scols).astype(
                 jnp.bfloat16)
    g_tgt = ((col3 == (et + base)[:, :, None]).astype(jnp.float32)
             * mask[:, :, None]).reshape(nsub, srows, scols).astype(
                 jnp.bfloat16)

    rcol = lax.broadcasted_iota(jnp.int32, (bb, max_e, n_rel), 2)
    r_oh = (rcol == er[:, :, None]).astype(jnp.float32).reshape(rows, n_rel)
    rel_c = (r_oh @ rel_ref[...]) @ w1b + mb1

    batch_dn = (((2,), (1,)), ((0,), (0,)))
    scat_dn = (((1,), (1,)), ((0,), (0,)))

    mwu = mw2 @ u1b
    mbu = mb2 @ u1b

    mbu_rows = jnp.broadcast_to(mbu[None], (nsub, srows, 2 * d))
    ub_all = ub1 + lax.dot_general(
        g_tgt, mbu_rows, scat_dn,
        preferred_element_type=jnp.float32).reshape(cols, 2 * d)

    s = jnp.broadcast_to(ent_ref[...][None], (bb, n_ent, d)).reshape(cols, d)
    for _ in range(_N_STEPS):
        sa = (s @ w1a).reshape(nsub, scols, 2 * d)
        sc = (s @ w1c).reshape(nsub, scols, 2 * d)
        hs = lax.dot_general(g_src, sa, batch_dn,
                             preferred_element_type=jnp.float32)
        ht = lax.dot_general(g_tgt, sc, batch_dn,
                             preferred_element_type=jnp.float32)
        h = _gelu_y(hs.reshape(rows, 2 * d) + ht.reshape(rows, 2 * d)
                    + rel_c)
        hm = (h @ mwu).reshape(nsub, srows, 2 * d)
        aggu = lax.dot_general(g_tgt, hm, scat_dn,
                               preferred_element_type=jnp.float32)
        u = _gelu_y(s @ u1a + aggu.reshape(cols, 2 * d) + ub_all)
        s = s + u @ uw2 + ub2

    qs = qs_ref[0]
    qt = qt_ref[0]
    qbase = lax.broadcasted_iota(jnp.int32, (bb, 1), 0) * n_ent
    qcol = lax.broadcasted_iota(jnp.int32, (bb, cols), 1)
    q_s = (qcol == (qs + qbase)).astype(jnp.float32)
    q_t = (qcol == (qt + qbase)).astype(jnp.float32)
    sv = q_s @ s
    tv = q_t @ s
    cw1 = cw1_ref[...] * rs2
    c = _gelu_y(sv @ cw1[:d, :] + tv @ cw1[d:, :] + cb1_ref[...] * rs2)
    out_ref[...] = c @ cw2_ref[...] + cb2_ref[...]


def kernel(edge_src, edge_rel, edge_tgt, n_edges, query_src, query_tgt,
           entity_table, rel_table,
           msg_w1, msg_b1, msg_w2, msg_b2,
           upd_w1, upd_b1, upd_w2, upd_b2,
           cls_w1, cls_b1, cls_w2, cls_b2):
    b, max_e = edge_src.shape
    bb = _BB
    nb = b // bb
    n_rel = cls_w2.shape[1]

    es = edge_src.reshape(nb, bb, max_e)
    er = edge_rel.reshape(nb, bb, max_e)
    et = edge_tgt.reshape(nb, bb, max_e)
    ne = n_edges.reshape(nb, bb, 1)
    qs = query_src.reshape(nb, bb, 1)
    qt = query_tgt.reshape(nb, bb, 1)

    mb1 = msg_b1.reshape(1, -1)
    mb2 = msg_b2.reshape(1, -1)
    ub1 = upd_b1.reshape(1, -1)
    ub2 = upd_b2.reshape(1, -1)
    cb1 = cls_b1.reshape(1, -1)
    cb2 = cls_b2.reshape(1, -1)

    def edge_spec():
        return pl.BlockSpec((1, bb, max_e), lambda i: (i, 0, 0))

    def scalar_spec():
        return pl.BlockSpec((1, bb, 1), lambda i: (i, 0, 0))

    def full_spec(a):
        nd = a.ndim
        return pl.BlockSpec(a.shape, lambda i: (0,) * nd)

    return pl.pallas_call(
        _gnn_block_kernel,
        grid=(nb,),
        in_specs=[
            edge_spec(), edge_spec(), edge_spec(),
            scalar_spec(), scalar_spec(), scalar_spec(),
            full_spec(entity_table), full_spec(rel_table),
            full_spec(msg_w1), full_spec(mb1), full_spec(msg_w2),
            full_spec(mb2),
            full_spec(upd_w1), full_spec(ub1), full_spec(upd_w2),
            full_spec(ub2),
            full_spec(cls_w1), full_spec(cb1), full_spec(cls_w2),
            full_spec(cb2),
        ],
        out_specs=pl.BlockSpec((bb, n_rel), lambda i: (i, 0)),
        out_shape=jax.ShapeDtypeStruct((b, n_rel), jnp.float32),
    )(es, er, et, ne, qs, qt,
      entity_table, rel_table,
      msg_w1, mb1, msg_w2, mb2,
      upd_w1, ub1, upd_w2, ub2,
      cls_w1, cb1, cls_w2, cb2)


# speedup vs baseline: 1.0266x; 1.0266x over previous
import math

import jax
import jax.numpy as jnp
from jax import lax
from jax.experimental import pallas as pl

_BB = 64
_SG = 4
_N_STEPS = 8


def _gelu(x):
    return 0.5 * x * (1.0 + lax.erf(x * (1.0 / math.sqrt(2.0))))


def _gnn_block_kernel(es_ref, er_ref, et_ref, ne_ref, qs_ref, qt_ref,
                      ent_ref, rel_ref,
                      mw1_ref, mb1_ref, mw2_ref, mb2_ref,
                      uw1_ref, ub1_ref, uw2_ref, ub2_ref,
                      cw1_ref, cb1_ref, cw2_ref, cb2_ref,
                      out_ref):
    n_ent, d = ent_ref.shape
    max_e = es_ref.shape[2]
    n_rel = rel_ref.shape[0]
    bb = _BB
    sg = _SG
    nsub = bb // sg
    srows = sg * max_e
    scols = sg * n_ent
    rows = bb * max_e
    cols = bb * n_ent

    w1 = mw1_ref[...]
    w1a = w1[:d, :]
    w1b = w1[d:2 * d, :]
    w1c = w1[2 * d:, :]
    mb1 = mb1_ref[...]
    mw2 = mw2_ref[...]
    mb2 = mb2_ref[...]
    uw1 = uw1_ref[...]
    u1a = uw1[:d, :]
    u1b = uw1[d:, :]
    ub1 = ub1_ref[...]
    uw2 = uw2_ref[...]
    ub2 = ub2_ref[...]

    es = es_ref[0]
    et = et_ref[0]
    er = er_ref[0]
    ne = ne_ref[0]
    eidx = lax.broadcasted_iota(jnp.int32, (bb, max_e), 1)
    mask = (eidx < ne).astype(jnp.float32)
    base = (lax.broadcasted_iota(jnp.int32, (bb, max_e), 0) % sg) * n_ent
    col3 = lax.broadcasted_iota(jnp.int32, (bb, max_e, scols), 2)
    g_src = ((col3 == (es + base)[:, :, None]).astype(jnp.float32)
             * mask[:, :, None]).reshape(nsub, srows, scols).astype(
                 jnp.bfloat16)
    g_tgt = ((col3 == (et + base)[:, :, None]).astype(jnp.float32)
             * mask[:, :, None]).reshape(nsub, srows, scols).astype(
                 jnp.bfloat16)

    rcol = lax.broadcasted_iota(jnp.int32, (bb, max_e, n_rel), 2)
    r_oh = (rcol == er[:, :, None]).astype(jnp.float32).reshape(rows, n_rel)
    rel_c = (r_oh @ rel_ref[...]) @ w1b + mb1

    batch_dn = (((2,), (1,)), ((0,), (0,)))
    scat_dn = (((1,), (1,)), ((0,), (0,)))

    mwu = mw2 @ u1b
    mbu = mb2 @ u1b

    s = jnp.broadcast_to(ent_ref[...][None], (bb, n_ent, d)).reshape(cols, d)
    for _ in range(_N_STEPS):
        sa = (s @ w1a).reshape(nsub, scols, 2 * d)
        sc = (s @ w1c).reshape(nsub, scols, 2 * d)
        hs = lax.dot_general(g_src, sa, batch_dn,
                             preferred_element_type=jnp.float32)
        ht = lax.dot_general(g_tgt, sc, batch_dn,
                             preferred_element_type=jnp.float32)
        h = _gelu(hs.reshape(rows, 2 * d) + ht.reshape(rows, 2 * d) + rel_c)
        hm = (h @ mwu + mbu).reshape(nsub, srows, 2 * d)
        aggu = lax.dot_general(g_tgt, hm, scat_dn,
                               preferred_element_type=jnp.float32)
        u = _gelu(s @ u1a + aggu.reshape(cols, 2 * d) + ub1)
        s = s + u @ uw2 + ub2

    qs = qs_ref[0]
    qt = qt_ref[0]
    qbase = lax.broadcasted_iota(jnp.int32, (bb, 1), 0) * n_ent
    qcol = lax.broadcasted_iota(jnp.int32, (bb, cols), 1)
    q_s = (qcol == (qs + qbase)).astype(jnp.float32)
    q_t = (qcol == (qt + qbase)).astype(jnp.float32)
    sv = q_s @ s
    tv = q_t @ s
    cw1 = cw1_ref[...]
    c = _gelu(sv @ cw1[:d, :] + tv @ cw1[d:, :] + cb1_ref[...])
    out_ref[...] = c @ cw2_ref[...] + cb2_ref[...]


def kernel(edge_src, edge_rel, edge_tgt, n_edges, query_src, query_tgt,
           entity_table, rel_table,
           msg_w1, msg_b1, msg_w2, msg_b2,
           upd_w1, upd_b1, upd_w2, upd_b2,
           cls_w1, cls_b1, cls_w2, cls_b2):
    b, max_e = edge_src.shape
    bb = _BB
    nb = b // bb
    n_rel = cls_w2.shape[1]

    es = edge_src.reshape(nb, bb, max_e)
    er = edge_rel.reshape(nb, bb, max_e)
    et = edge_tgt.reshape(nb, bb, max_e)
    ne = n_edges.reshape(nb, bb, 1)
    qs = query_src.reshape(nb, bb, 1)
    qt = query_tgt.reshape(nb, bb, 1)

    mb1 = msg_b1.reshape(1, -1)
    mb2 = msg_b2.reshape(1, -1)
    ub1 = upd_b1.reshape(1, -1)
    ub2 = upd_b2.reshape(1, -1)
    cb1 = cls_b1.reshape(1, -1)
    cb2 = cls_b2.reshape(1, -1)

    def edge_spec():
        return pl.BlockSpec((1, bb, max_e), lambda i: (i, 0, 0))

    def scalar_spec():
        return pl.BlockSpec((1, bb, 1), lambda i: (i, 0, 0))

    def full_spec(a):
        nd = a.ndim
        return pl.BlockSpec(a.shape, lambda i: (0,) * nd)

    return pl.pallas_call(
        _gnn_block_kernel,
        grid=(nb,),
        in_specs=[
            edge_spec(), edge_spec(), edge_spec(),
            scalar_spec(), scalar_spec(), scalar_spec(),
            full_spec(entity_table), full_spec(rel_table),
            full_spec(msg_w1), full_spec(mb1), full_spec(msg_w2),
            full_spec(mb2),
            full_spec(upd_w1), full_spec(ub1), full_spec(upd_w2),
            full_spec(ub2),
            full_spec(cls_w1), full_spec(cb1), full_spec(cls_w2),
            full_spec(cb2),
        ],
        out_specs=pl.BlockSpec((bb, n_rel), lambda i: (i, 0)),
        out_shape=jax.ShapeDtypeStruct((b, n_rel), jnp.float32),
    )(es, er, et, ne, qs, qt,
      entity_table, rel_table,
      msg_w1, mb1, msg_w2, mb2,
      upd_w1, ub1, upd_w2, ub2,
      cls_w1, cb1, cls_w2, cb2)
